# baseline (speedup 1.0000x reference)
import functools

import jax
import jax.numpy as jnp
from jax.experimental import pallas as pl
from jax.experimental.pallas import tpu as pltpu

_K = 2


def _gelu_exact(v):
    return 0.5 * v * (1.0 + jax.lax.erf(v * (2.0 ** -0.5)))


def _moe_body(x_ref, w_ref, W1_ref, W2_ref, b1_ref, Ws1_ref, Ws2_ref,
              bs1_ref, out_ref, *, Bt, E):
    e, f, t = pl.program_id(0), pl.program_id(1), pl.program_id(2)
    rows = pl.ds(t * Bt, Bt)
    x = x_ref[rows, :]

    def ffn(W1b, b1v, W2b):
        h = jax.lax.dot_general(x, W1b, (((1,), (1,)), ((), ())),
                                preferred_element_type=jnp.float32)
        h = _gelu_exact(h + b1v)
        return jax.lax.dot_general(h, W2b, (((1,), (1,)), ((), ())),
                                   preferred_element_type=jnp.float32)

    @pl.when(e < E)
    def _routed():
        y = ffn(W1_ref[0], b1_ref[0], W2_ref[0])
        contrib = w_ref[0] * y

        @pl.when((e == 0) & (f == 0))
        def _():
            out_ref[rows, :] = contrib

        @pl.when((e > 0) | (f > 0))
        def _():
            out_ref[rows, :] = out_ref[rows, :] + contrib

    @pl.when(e == E)
    def _shared():
        y = ffn(Ws1_ref[...], bs1_ref[0], Ws2_ref[...])
        out_ref[rows, :] = out_ref[rows, :] + y


def _moe(x2, w3, W1, W2, b1r, Ws1, Ws2, bs1r, *, interpret=False):
    S, D = x2.shape
    E, F, _ = W1.shape
    Bt = min(512, S)
    T = S // Bt
    F2 = 4 if F % 4 == 0 and F >= 512 else 1
    Fc = F // F2
    grid = (E + 1, F2, T)

    def w1_map(e, f, t):
        return (jnp.minimum(e, E - 1), jnp.where(e < E, f, F2 - 1), 0)

    def w2_map(e, f, t):
        return (jnp.minimum(e, E - 1), 0, jnp.where(e < E, f, F2 - 1))

    return pl.pallas_call(
        functools.partial(_moe_body, Bt=Bt, E=E),
        grid=grid,
        in_specs=[
            pl.BlockSpec((S, D), lambda e, f, t: (0, 0)),
            pl.BlockSpec((1, Bt, 1), lambda e, f, t: (jnp.minimum(e, E - 1),
                                                      t, 0)),
            pl.BlockSpec((1, Fc, D), w1_map),
            pl.BlockSpec((1, D, Fc), w2_map),
            pl.BlockSpec((1, 1, Fc), lambda e, f, t: (jnp.minimum(e, E - 1),
                                                      0,
                                                      jnp.where(e < E, f, 0))),
            pl.BlockSpec((Fc, D), lambda e, f, t: (jnp.where(e == E, f, 0), 0)),
            pl.BlockSpec((D, Fc), lambda e, f, t: (0, jnp.where(e == E, f, 0))),
            pl.BlockSpec((1, Fc), lambda e, f, t: (0, jnp.where(e == E, f, 0))),
        ],
        out_specs=pl.BlockSpec((S, D), lambda e, f, t: (0, 0)),
        out_shape=jax.ShapeDtypeStruct((S, D), jnp.float32),
        compiler_params=pltpu.CompilerParams(
            dimension_semantics=("arbitrary", "arbitrary", "arbitrary"),
        ),
        interpret=interpret,
    )(x2, w3, W1, W2, b1r, Ws1, Ws2, bs1r)


def kernel(x, gate_w, W1, b1, W2, b2, Ws1, bs1, Ws2, bs2, route_scale,
           *, interpret=False):
    original_shape = x.shape
    if x.ndim == 2:
        x = x[:, None, :]
    Bx, Sx, D = x.shape
    E, F, _ = W1.shape

    gate_scores = x @ gate_w.T
    scores = jax.nn.sigmoid(gate_scores) * route_scale
    topk_scores, topk_idx = jax.lax.top_k(scores, _K)
    topk_w = topk_scores / jnp.sum(topk_scores, axis=-1, keepdims=True)

    onehot = jax.nn.one_hot(topk_idx, E, dtype=jnp.float32)
    w_full = jnp.einsum("bske,bsk->bse", onehot, topk_w)
    present = jnp.any(onehot > 0, axis=(0, 1))
    counts = jnp.sum(present.astype(jnp.float32), axis=0)
    expert_usage = counts / jnp.sum(counts)

    S = Bx * Sx
    x2 = x.reshape(S, D)
    w3 = w_full.reshape(S, E).T[:, :, None]
    b1r = b1[:, None, :]
    bs1r = bs1[None, :]

    out = _moe(x2, w3, W1, W2, b1r, Ws1, Ws2, bs1r, interpret=interpret)
    out = out + w_full.reshape(S, E) @ b2 + bs2[None, :]
    output = out.reshape(original_shape)
    return output, expert_usage, topk_idx


# speedup vs baseline: 2.0178x; 2.0178x over previous
import functools

import jax
import jax.numpy as jnp
from jax.experimental import pallas as pl
from jax.experimental.pallas import tpu as pltpu

_K = 2


def _gelu_exact(v):
    return 0.5 * v * (1.0 + jax.lax.erf(v * (2.0 ** -0.5)))


def _routed_body(x_ref, w_ref, W1_ref, W2_ref, b1_ref, out_ref):
    e = pl.program_id(1)
    x = x_ref[...]
    h = jax.lax.dot_general(x, W1_ref[0], (((1,), (1,)), ((), ())),
                            preferred_element_type=jnp.float32)
    h = _gelu_exact(h + b1_ref[0])
    y = jax.lax.dot_general(h, W2_ref[0], (((1,), (1,)), ((), ())),
                            preferred_element_type=jnp.float32)
    contrib = w_ref[0] * y

    @pl.when(e == 0)
    def _():
        out_ref[...] = contrib

    @pl.when(e > 0)
    def _():
        out_ref[...] = out_ref[...] + contrib


def _shared_body(x_ref, acc_ref, Ws1_ref, Ws2_ref, bs1_ref, out_ref):
    x = x_ref[...]
    h = jax.lax.dot_general(x, Ws1_ref[...], (((1,), (1,)), ((), ())),
                            preferred_element_type=jnp.float32)
    h = _gelu_exact(h + bs1_ref[...])
    y = jax.lax.dot_general(h, Ws2_ref[...], (((1,), (1,)), ((), ())),
                            preferred_element_type=jnp.float32)
    out_ref[...] = acc_ref[...] + y


def _moe(x2, w3, W1, W2, b1r, Ws1, Ws2, bs1r, *, interpret=False):
    S, D = x2.shape
    E, F, _ = W1.shape
    Bt = min(512, S)
    T = S // Bt

    acc = pl.pallas_call(
        _routed_body,
        grid=(T, E),
        in_specs=[
            pl.BlockSpec((Bt, D), lambda t, e: (t, 0)),
            pl.BlockSpec((1, Bt, 1), lambda t, e: (e, t, 0)),
            pl.BlockSpec((1, F, D), lambda t, e: (e, 0, 0)),
            pl.BlockSpec((1, D, F), lambda t, e: (e, 0, 0)),
            pl.BlockSpec((1, 1, F), lambda t, e: (e, 0, 0)),
        ],
        out_specs=pl.BlockSpec((Bt, D), lambda t, e: (t, 0)),
        out_shape=jax.ShapeDtypeStruct((S, D), jnp.float32),
        compiler_params=pltpu.CompilerParams(
            dimension_semantics=("parallel", "arbitrary"),
        ),
        interpret=interpret,
    )(x2, w3, W1, W2, b1r)

    out = pl.pallas_call(
        _shared_body,
        grid=(T,),
        in_specs=[
            pl.BlockSpec((Bt, D), lambda t: (t, 0)),
            pl.BlockSpec((Bt, D), lambda t: (t, 0)),
            pl.BlockSpec((F, D), lambda t: (0, 0)),
            pl.BlockSpec((D, F), lambda t: (0, 0)),
            pl.BlockSpec((1, F), lambda t: (0, 0)),
        ],
        out_specs=pl.BlockSpec((Bt, D), lambda t: (t, 0)),
        out_shape=jax.ShapeDtypeStruct((S, D), jnp.float32),
        input_output_aliases={1: 0},
        compiler_params=pltpu.CompilerParams(
            dimension_semantics=("parallel",),
        ),
        interpret=interpret,
    )(x2, acc, Ws1, Ws2, bs1r)
    return out


def kernel(x, gate_w, W1, b1, W2, b2, Ws1, bs1, Ws2, bs2, route_scale,
           *, interpret=False):
    original_shape = x.shape
    if x.ndim == 2:
        x = x[:, None, :]
    Bx, Sx, D = x.shape
    E, F, _ = W1.shape

    gate_scores = x @ gate_w.T
    scores = jax.nn.sigmoid(gate_scores) * route_scale
    topk_scores, topk_idx = jax.lax.top_k(scores, _K)
    topk_w = topk_scores / jnp.sum(topk_scores, axis=-1, keepdims=True)

    onehot = jax.nn.one_hot(topk_idx, E, dtype=jnp.float32)
    w_full = jnp.einsum("bske,bsk->bse", onehot, topk_w)
    present = jnp.any(onehot > 0, axis=(0, 1))
    counts = jnp.sum(present.astype(jnp.float32), axis=0)
    expert_usage = counts / jnp.sum(counts)

    S = Bx * Sx
    x2 = x.reshape(S, D)
    w3 = w_full.reshape(S, E).T[:, :, None]
    b1r = b1[:, None, :]
    bs1r = bs1[None, :]

    out = _moe(x2, w3, W1, W2, b1r, Ws1, Ws2, bs1r, interpret=interpret)
    out = out + w_full.reshape(S, E) @ b2 + bs2[None, :]
    output = out.reshape(original_shape)
    return output, expert_usage, topk_idx
